# Initial kernel scaffold; baseline (speedup 1.0000x reference)
import jax
import jax.numpy as jnp
from jax.experimental import pallas as pl


def kernel(ind_2, p1, p3, p5, diff, basis, pp1_w1, pp1_b1, pp1_w2, pp1_b2, pi1_w1, pi1_b1, pi1_w2, pi1_b2, ii1_w1, ii1_w2, pp3_w1, pp3_w2, ii3_w1, ii3_w2, pp5_w1, pp5_b1, pp5_w2, pp5_b2, ii5_w1, ii5_w2):
    raise NotImplementedError("write your pallas kernel here")


# speedup vs baseline: 20.4721x; 20.4721x over previous
import functools

import jax
import jax.numpy as jnp
from jax import lax
from jax.experimental import pallas as pl
from jax.experimental.pallas import tpu as pltpu
from jax.experimental.pallas import tpu_sc as plsc

N = 10000
E = 160000
D = 16
NB = 10

NC = 2
NS = 16
NW = NC * NS
CH = 128
EPW = 5120
E_PAD = NW * EPW
N_PAD = 10240
AW = 128
TW = 256
MW = 128


def _k0_body(p1_ref, p3f_ref, p5f_ref, w11_ref, b11_ref, w12_ref, b12_ref,
             wa_ref, wb_ref, b1i_ref, w3a_ref, w3b_ref, w5a_ref, b5a_ref,
             w5b_ref, b5b_ref, w5c_ref, w5d_ref, a_ref, t_ref):
    blk = p1_ref.shape[0]
    pad = jnp.zeros((blk, AW - D), jnp.float32)
    p1h = jnp.tanh(p1_ref[...] @ w11_ref[...] + b11_ref[...])
    p1h = jnp.tanh(p1h @ w12_ref[...] + b12_ref[...])
    a1 = p1h @ wa_ref[...]
    a2 = p1h @ wb_ref[...] + b1i_ref[...]
    a_ref[...] = jnp.concatenate([a1, pad], axis=1)
    q3 = jnp.tanh(jnp.tanh(p3f_ref[...] @ w3a_ref[...]) @ w3b_ref[...])
    p5h = jnp.tanh(p5f_ref[...] @ w5a_ref[...] + b5a_ref[...])
    p5h = jnp.tanh(p5h @ w5b_ref[...] + b5b_ref[...])
    q5 = jnp.tanh(jnp.tanh(p5h @ w5c_ref[...]) @ w5d_ref[...])
    t_ref[...] = jnp.concatenate([a2, q3, q5, pad[:, :TW - 13 * D]], axis=1)


def _node_precompute(p1, p3f, p5f, w11, b11, w12, b12, wa, wb, b1i, w3a, w3b,
                     w5a, b5a, w5b, b5b, w5c, w5d):
    blk = 2000
    grid = N // blk
    row = lambda i: (i, 0)
    full = lambda a: pl.BlockSpec(a.shape, lambda i: (0, 0))
    return pl.pallas_call(
        _k0_body,
        grid=(grid,),
        in_specs=[
            pl.BlockSpec((blk, D), row),
            pl.BlockSpec((blk, 3 * D), row),
            pl.BlockSpec((blk, 9 * D), row),
            full(w11), full(b11), full(w12), full(b12), full(wa), full(wb),
            full(b1i), full(w3a), full(w3b), full(w5a), full(b5a), full(w5b),
            full(b5b), full(w5c), full(w5d),
        ],
        out_specs=[
            pl.BlockSpec((blk, AW), row),
            pl.BlockSpec((blk, TW), row),
        ],
        out_shape=[
            jax.ShapeDtypeStruct((N, AW), jnp.float32),
            jax.ShapeDtypeStruct((N, TW), jnp.float32),
        ],
    )(p1, p3f, p5f, w11, b11, w12, b12, wa, wb, b1i, w3a, w3b, w5a, b5a,
      w5b, b5b, w5c, w5d)


def _gather_body(ta, tt, indi, indj, ga, gt, idxi_v, idxj_v, ba, bt, s1, s2):
    wid = lax.axis_index("s") * NC + lax.axis_index("c")
    base0 = wid * EPW

    def step(g, carry):
        base = base0 + g * CH
        pltpu.sync_copy(indi.at[pl.ds(base, CH)], idxi_v)
        pltpu.sync_copy(indj.at[pl.ds(base, CH)], idxj_v)
        c1 = pltpu.async_copy(ta.at[idxi_v], ba, s1)
        c2 = pltpu.async_copy(tt.at[idxj_v], bt, s2)
        c1.wait()
        c2.wait()
        pltpu.sync_copy(ba, ga.at[pl.ds(base, CH)])
        pltpu.sync_copy(bt, gt.at[pl.ds(base, CH)])
        return carry

    lax.fori_loop(0, EPW // CH, step, 0)


def _sc_gather(ta, tt, indi, indj):
    mesh = plsc.VectorSubcoreMesh(core_axis_name="c", subcore_axis_name="s")
    fn = functools.partial(
        pl.kernel,
        mesh=mesh,
        out_type=[
            jax.ShapeDtypeStruct((E_PAD, AW), jnp.float32),
            jax.ShapeDtypeStruct((E_PAD, TW), jnp.float32),
        ],
        scratch_types=[
            pltpu.VMEM((CH,), jnp.int32),
            pltpu.VMEM((CH,), jnp.int32),
            pltpu.VMEM((CH, AW), jnp.float32),
            pltpu.VMEM((CH, TW), jnp.float32),
            pltpu.SemaphoreType.DMA,
            pltpu.SemaphoreType.DMA,
        ],
    )(_gather_body)
    return fn(ta, tt, indi, indj)


def _k2_body(ga_ref, gt_ref, dp_ref, w2_ref, b2_ref, w1f_ref, wi2_ref,
             m0_ref, m1_ref):
    gt = gt_ref[...]
    inter = jnp.tanh(ga_ref[:, 0:16] + gt[:, 0:16])
    inter2 = jnp.tanh(inter @ w2_ref[...] + b2_ref[...])
    basis = dp_ref[:, 3:13]
    brep = jnp.concatenate([basis] * D, axis=1)
    h = jnp.tanh((inter2 * brep) @ w1f_ref[...])
    i1 = jnp.tanh(h @ wi2_ref[...])
    i1_1 = i1[:, 0:16]
    i1_2 = i1[:, 16:32]
    i1_3 = i1[:, 32:48]
    i1_4 = i1[:, 48:64]
    sd = [dp_ref[:, a:a + 1] * i1_2 for a in range(3)]
    m3 = [gt[:, 16 + 16 * a:32 + 16 * a] * i1_3 + sd[a] for a in range(3)]
    m5 = [gt[:, 64 + 16 * ab:80 + 16 * ab] * i1_4
          + sd[ab // 3] * sd[ab % 3] for ab in range(9)]
    zpad = jnp.zeros((i1.shape[0], MW - 7 * D), jnp.float32)
    m0_ref[...] = jnp.concatenate([i1_1] + m3 + m5[0:3] + [zpad], axis=1)
    m1_ref[...] = jnp.concatenate(m5[3:9] + [zpad, zpad], axis=1)


def _edge_mlp(ga, gt, dp, w2, b2, w1f, wi2):
    blk = 2048
    grid = E_PAD // blk
    row = lambda i: (i, 0)
    full = lambda a: pl.BlockSpec(a.shape, lambda i: (0, 0))
    return pl.pallas_call(
        _k2_body,
        grid=(grid,),
        in_specs=[
            pl.BlockSpec((blk, AW), row),
            pl.BlockSpec((blk, TW), row),
            pl.BlockSpec((blk, D), row),
            full(w2), full(b2), full(w1f), full(wi2),
        ],
        out_specs=[
            pl.BlockSpec((blk, MW), row),
            pl.BlockSpec((blk, MW), row),
        ],
        out_shape=[
            jax.ShapeDtypeStruct((E_PAD, MW), jnp.float32),
            jax.ShapeDtypeStruct((E_PAD, MW), jnp.float32),
        ],
    )(ga, gt, dp, w2, b2, w1f, wi2)


def _scatter_body(m0, m1, inds, zeros, out, idx_v, mbuf, acc, sem):
    c = lax.axis_index("c")
    s = lax.axis_index("s")

    @pl.when(s == 0)
    def _():
        pltpu.sync_copy(zeros, acc)

    plsc.subcore_barrier()

    def make_step(m_ref):
        def step(g, carry):
            base = s * (E_PAD // NS) + g * CH
            pltpu.sync_copy(inds.at[pl.ds(base, CH)], idx_v)
            pltpu.sync_copy(m_ref.at[pl.ds(base, CH)], mbuf)
            pltpu.sync_copy(mbuf, acc.at[idx_v], add=True)
            return carry
        return step

    nchunks = E_PAD // NS // CH

    @pl.when(c == 0)
    def _():
        lax.fori_loop(0, nchunks, make_step(m0), 0)

    @pl.when(c == 1)
    def _():
        lax.fori_loop(0, nchunks, make_step(m1), 0)

    plsc.subcore_barrier()
    rows = N_PAD // NS
    pltpu.sync_copy(acc.at[pl.ds(s * rows, rows)],
                    out.at[pl.ds(c * N_PAD + s * rows, rows)])


def _sc_scatter(m0, m1, inds, zeros):
    mesh = plsc.VectorSubcoreMesh(core_axis_name="c", subcore_axis_name="s")
    fn = functools.partial(
        pl.kernel,
        mesh=mesh,
        out_type=jax.ShapeDtypeStruct((2 * N_PAD, MW), jnp.float32),
        scratch_types=[
            pltpu.VMEM((CH,), jnp.int32),
            pltpu.VMEM((CH, MW), jnp.float32),
            pltpu.VMEM_SHARED((N_PAD, MW), jnp.float32),
            pltpu.SemaphoreType.DMA,
        ],
    )(_scatter_body)
    return fn(m0, m1, inds, zeros)


def _k4_body(a0_ref, a1_ref, o1_ref, o3_ref, o5_ref):
    a0 = a0_ref[...]
    a1 = a1_ref[...]
    p1o = a0[:, 0:16]
    p3o = [a0[:, 16 + 16 * a:32 + 16 * a] for a in range(3)]
    p5o = [a0[:, 64 + 16 * ab:80 + 16 * ab] for ab in range(3)] + \
          [a1[:, 16 * (ab - 3):16 * (ab - 2)] for ab in range(3, 9)]
    t = p1o
    for a in range(3):
        for b in range(3):
            t = t + p3o[a] * p5o[3 * a + b] * p3o[b]
    o1_ref[...] = t
    o3_ref[...] = jnp.concatenate([p3o[a] * t for a in range(3)], axis=1)
    o5_ref[...] = jnp.concatenate([p5o[ab] * t for ab in range(9)], axis=1)


def _combine(acc0, acc1):
    blk = 2000
    grid = N // blk
    row = lambda i: (i, 0)
    return pl.pallas_call(
        _k4_body,
        grid=(grid,),
        in_specs=[
            pl.BlockSpec((blk, MW), row),
            pl.BlockSpec((blk, MW), row),
        ],
        out_specs=[
            pl.BlockSpec((blk, D), row),
            pl.BlockSpec((blk, 3 * D), row),
            pl.BlockSpec((blk, 9 * D), row),
        ],
        out_shape=[
            jax.ShapeDtypeStruct((N, D), jnp.float32),
            jax.ShapeDtypeStruct((N, 3 * D), jnp.float32),
            jax.ShapeDtypeStruct((N, 9 * D), jnp.float32),
        ],
    )(acc0, acc1)


def kernel(ind_2, p1, p3, p5, diff, basis, pp1_w1, pp1_b1, pp1_w2, pp1_b2,
           pi1_w1, pi1_b1, pi1_w2, pi1_b2, ii1_w1, ii1_w2, pp3_w1, pp3_w2,
           ii3_w1, ii3_w2, pp5_w1, pp5_b1, pp5_w2, pp5_b2, ii5_w1, ii5_w2):
    f32 = jnp.float32
    eye3 = jnp.eye(3, dtype=f32)
    eye9 = jnp.eye(9, dtype=f32)

    wa = pi1_w1[:D]
    wb = pi1_w1[D:]
    w3f = pp3_w1 @ pp3_w2 @ ii3_w1
    w3a = jnp.kron(eye3, w3f)
    w3b = jnp.kron(eye3, ii3_w2)
    w5a = jnp.kron(eye9, pp5_w1)
    b5a = jnp.tile(pp5_b1, 9)[None, :]
    w5b = jnp.kron(eye9, pp5_w2)
    b5b = jnp.tile(pp5_b2, 9)[None, :]
    w5c = jnp.kron(eye9, ii5_w1)
    w5d = jnp.kron(eye9, ii5_w2)
    w1f = jnp.repeat(ii1_w1, NB, axis=0)

    b11 = pp1_b1[None, :]
    b12 = pp1_b2[None, :]
    b1i = pi1_b1[None, :]
    b2 = pi1_b2[None, :]

    p3f = p3.reshape(N, 3 * D)
    p5f = p5.reshape(N, 9 * D)

    ta, tt = _node_precompute(p1, p3f, p5f, pp1_w1, b11, pp1_w2, b12, wa, wb,
                              b1i, w3a, w3b, w5a, b5a, w5b, b5b, w5c, w5d)

    pad = E_PAD - E
    ind_i = ind_2[:, 0]
    ind_j = ind_2[:, 1]
    indi = jnp.concatenate([ind_i, jnp.zeros((pad,), jnp.int32)])
    indj = jnp.concatenate([ind_j, jnp.zeros((pad,), jnp.int32)])
    inds = jnp.concatenate([ind_i, jnp.full((pad,), N, jnp.int32)])
    dp = jnp.concatenate([diff, basis, jnp.zeros((E, 3), f32)], axis=1)
    dp = jnp.concatenate([dp, jnp.zeros((pad, D), f32)], axis=0)

    ga, gt = _sc_gather(ta, tt, indi, indj)
    m0, m1 = _edge_mlp(ga, gt, dp, pi1_w2, b2, w1f, ii1_w2)
    zeros = jnp.zeros((N_PAD, MW), f32)
    acc = _sc_scatter(m0, m1, inds, zeros)
    o1, o3, o5 = _combine(acc[:N], acc[N_PAD:N_PAD + N])
    return (o1, o3.reshape(N, 3, D), o5.reshape(N, 3, 3, D))


# speedup vs baseline: 23.3764x; 1.1419x over previous
import functools

import jax
import jax.numpy as jnp
from jax import lax
from jax.experimental import pallas as pl
from jax.experimental.pallas import tpu as pltpu
from jax.experimental.pallas import tpu_sc as plsc

N = 10000
E = 160000
D = 16
NB = 10

NC = 2
NS = 16
NW = NC * NS
CH = 128
EPW = 5120
E_PAD = NW * EPW
N_PAD = 10240
AW = 128
TW = 256
MW = 128


def _k0_body(p1_ref, p3f_ref, p5f_ref, w11_ref, b11_ref, w12_ref, b12_ref,
             wa_ref, wb_ref, b1i_ref, w3a_ref, w3b_ref, w5a_ref, b5a_ref,
             w5b_ref, b5b_ref, w5c_ref, w5d_ref, a_ref, t_ref):
    blk = p1_ref.shape[0]
    pad = jnp.zeros((blk, AW - D), jnp.float32)
    p1h = jnp.tanh(p1_ref[...] @ w11_ref[...] + b11_ref[...])
    p1h = jnp.tanh(p1h @ w12_ref[...] + b12_ref[...])
    a1 = p1h @ wa_ref[...]
    a2 = p1h @ wb_ref[...] + b1i_ref[...]
    a_ref[...] = jnp.concatenate([a1, pad], axis=1)
    q3 = jnp.tanh(jnp.tanh(p3f_ref[...] @ w3a_ref[...]) @ w3b_ref[...])
    p5h = jnp.tanh(p5f_ref[...] @ w5a_ref[...] + b5a_ref[...])
    p5h = jnp.tanh(p5h @ w5b_ref[...] + b5b_ref[...])
    q5 = jnp.tanh(jnp.tanh(p5h @ w5c_ref[...]) @ w5d_ref[...])
    t_ref[...] = jnp.concatenate([a2, q3, q5, pad[:, :TW - 13 * D]], axis=1)


def _node_precompute(p1, p3f, p5f, w11, b11, w12, b12, wa, wb, b1i, w3a, w3b,
                     w5a, b5a, w5b, b5b, w5c, w5d):
    blk = 2000
    grid = N // blk
    row = lambda i: (i, 0)
    full = lambda a: pl.BlockSpec(a.shape, lambda i: (0, 0))
    return pl.pallas_call(
        _k0_body,
        grid=(grid,),
        in_specs=[
            pl.BlockSpec((blk, D), row),
            pl.BlockSpec((blk, 3 * D), row),
            pl.BlockSpec((blk, 9 * D), row),
            full(w11), full(b11), full(w12), full(b12), full(wa), full(wb),
            full(b1i), full(w3a), full(w3b), full(w5a), full(b5a), full(w5b),
            full(b5b), full(w5c), full(w5d),
        ],
        out_specs=[
            pl.BlockSpec((blk, AW), row),
            pl.BlockSpec((blk, TW), row),
        ],
        out_shape=[
            jax.ShapeDtypeStruct((N, AW), jnp.float32),
            jax.ShapeDtypeStruct((N, TW), jnp.float32),
        ],
    )(p1, p3f, p5f, w11, b11, w12, b12, wa, wb, b1i, w3a, w3b, w5a, b5a,
      w5b, b5b, w5c, w5d)


NCH1 = EPW // CH


def _gather_body(ta, tt, indi2d, indj2d, gt,
                 idxi, idxj, ba0, ba1, bt0, bt1, sa0, sa1, st0, st1):
    wid = lax.axis_index("s") * NC + lax.axis_index("c")
    base0 = wid * EPW
    pltpu.sync_copy(indi2d.at[pl.ds(wid * NCH1, NCH1)], idxi)
    pltpu.sync_copy(indj2d.at[pl.ds(wid * NCH1, NCH1)], idxj)
    bas = [ba0, ba1]
    bts = [bt0, bt1]
    sas = [sa0, sa1]
    sts = [st0, st1]

    def fire(g):
        i = g % 2
        ha = pltpu.async_copy(ta.at[idxi.at[g]], bas[i], sas[i])
        ht = pltpu.async_copy(tt.at[idxj.at[g]], bts[i], sts[i])
        return ha, ht

    prev = fire(0)
    for g in range(NCH1):
        i = g % 2
        nxt = fire(g + 1) if g + 1 < NCH1 else None
        prev[0].wait()
        prev[1].wait()

        def edge_body(e8, carry):
            for k in range(8):
                e = e8 * 8 + k
                bts[i][e, 0:16] = bas[i][e, 0:16] + bts[i][e, 0:16]
            return carry

        lax.fori_loop(0, CH // 8, edge_body, 0)
        pltpu.sync_copy(bts[i], gt.at[pl.ds(base0 + g * CH, CH)])
        prev = nxt


def _sc_gather(ta, tt, indi2d, indj2d):
    mesh = plsc.VectorSubcoreMesh(core_axis_name="c", subcore_axis_name="s")
    fn = functools.partial(
        pl.kernel,
        mesh=mesh,
        out_type=jax.ShapeDtypeStruct((E_PAD, TW), jnp.float32),
        scratch_types=[
            pltpu.VMEM((NCH1, CH), jnp.int32),
            pltpu.VMEM((NCH1, CH), jnp.int32),
            pltpu.VMEM((CH, AW), jnp.float32),
            pltpu.VMEM((CH, AW), jnp.float32),
            pltpu.VMEM((CH, TW), jnp.float32),
            pltpu.VMEM((CH, TW), jnp.float32),
            pltpu.SemaphoreType.DMA,
            pltpu.SemaphoreType.DMA,
            pltpu.SemaphoreType.DMA,
            pltpu.SemaphoreType.DMA,
        ],
    )(_gather_body)
    return fn(ta, tt, indi2d, indj2d)


def _k2_body(gt_ref, dp_ref, w2_ref, b2_ref, w1f_ref, wi2_ref,
             m0_ref, m1_ref):
    gt = gt_ref[...]
    inter = jnp.tanh(gt[:, 0:16])
    inter2 = jnp.tanh(inter @ w2_ref[...] + b2_ref[...])
    basis = dp_ref[:, 3:13]
    brep = jnp.concatenate([basis] * D, axis=1)
    h = jnp.tanh((inter2 * brep) @ w1f_ref[...])
    i1 = jnp.tanh(h @ wi2_ref[...])
    i1_1 = i1[:, 0:16]
    i1_2 = i1[:, 16:32]
    i1_3 = i1[:, 32:48]
    i1_4 = i1[:, 48:64]
    sd = [dp_ref[:, a:a + 1] * i1_2 for a in range(3)]
    m3 = [gt[:, 16 + 16 * a:32 + 16 * a] * i1_3 + sd[a] for a in range(3)]
    m5 = [gt[:, 64 + 16 * ab:80 + 16 * ab] * i1_4
          + sd[ab // 3] * sd[ab % 3] for ab in range(9)]
    zpad = jnp.zeros((i1.shape[0], MW - 7 * D), jnp.float32)
    m0_ref[...] = jnp.concatenate([i1_1] + m3 + m5[0:3] + [zpad], axis=1)
    m1_ref[...] = jnp.concatenate(m5[3:9] + [zpad, zpad], axis=1)


def _edge_mlp(gt, dp, w2, b2, w1f, wi2):
    blk = 2048
    grid = E_PAD // blk
    row = lambda i: (i, 0)
    full = lambda a: pl.BlockSpec(a.shape, lambda i: (0, 0))
    return pl.pallas_call(
        _k2_body,
        grid=(grid,),
        in_specs=[
            pl.BlockSpec((blk, TW), row),
            pl.BlockSpec((blk, D), row),
            full(w2), full(b2), full(w1f), full(wi2),
        ],
        out_specs=[
            pl.BlockSpec((blk, MW), row),
            pl.BlockSpec((blk, MW), row),
        ],
        out_shape=[
            jax.ShapeDtypeStruct((E_PAD, MW), jnp.float32),
            jax.ShapeDtypeStruct((E_PAD, MW), jnp.float32),
        ],
    )(gt, dp, w2, b2, w1f, wi2)


NCH3 = E_PAD // NS // CH


def _scatter_body(m0, m1, inds2d, zeros, out, idx2d, mb0, mb1, acc, s0, s1):
    c = lax.axis_index("c")
    s = lax.axis_index("s")

    @pl.when(s == 0)
    def _():
        pltpu.sync_copy(zeros, acc)

    pltpu.sync_copy(inds2d.at[pl.ds(s * NCH3, NCH3)], idx2d)
    plsc.subcore_barrier()
    mbs = [mb0, mb1]
    sems = [s0, s1]

    def run(m_ref):
        def fire(g):
            base = s * (E_PAD // NS) + g * CH
            return pltpu.async_copy(m_ref.at[pl.ds(base, CH)],
                                    mbs[g % 2], sems[g % 2])

        prev = fire(0)
        for g in range(NCH3):
            nxt = fire(g + 1) if g + 1 < NCH3 else None
            prev.wait()
            pltpu.sync_copy(mbs[g % 2], acc.at[idx2d.at[g]], add=True)
            prev = nxt

    @pl.when(c == 0)
    def _():
        run(m0)

    @pl.when(c == 1)
    def _():
        run(m1)

    plsc.subcore_barrier()
    rows = N_PAD // NS
    pltpu.sync_copy(acc.at[pl.ds(s * rows, rows)],
                    out.at[pl.ds(c * N_PAD + s * rows, rows)])


def _sc_scatter(m0, m1, inds2d, zeros):
    mesh = plsc.VectorSubcoreMesh(core_axis_name="c", subcore_axis_name="s")
    fn = functools.partial(
        pl.kernel,
        mesh=mesh,
        out_type=jax.ShapeDtypeStruct((2 * N_PAD, MW), jnp.float32),
        scratch_types=[
            pltpu.VMEM((NCH3, CH), jnp.int32),
            pltpu.VMEM((CH, MW), jnp.float32),
            pltpu.VMEM((CH, MW), jnp.float32),
            pltpu.VMEM_SHARED((N_PAD, MW), jnp.float32),
            pltpu.SemaphoreType.DMA,
            pltpu.SemaphoreType.DMA,
        ],
    )(_scatter_body)
    return fn(m0, m1, inds2d, zeros)


def _k4_body(a0_ref, a1_ref, o1_ref, o3_ref, o5_ref):
    a0 = a0_ref[...]
    a1 = a1_ref[...]
    p1o = a0[:, 0:16]
    p3o = [a0[:, 16 + 16 * a:32 + 16 * a] for a in range(3)]
    p5o = [a0[:, 64 + 16 * ab:80 + 16 * ab] for ab in range(3)] + \
          [a1[:, 16 * (ab - 3):16 * (ab - 2)] for ab in range(3, 9)]
    t = p1o
    for a in range(3):
        for b in range(3):
            t = t + p3o[a] * p5o[3 * a + b] * p3o[b]
    o1_ref[...] = t
    o3_ref[...] = jnp.concatenate([p3o[a] * t for a in range(3)], axis=1)
    o5_ref[...] = jnp.concatenate([p5o[ab] * t for ab in range(9)], axis=1)


def _combine(acc0, acc1):
    blk = 2000
    grid = N // blk
    row = lambda i: (i, 0)
    return pl.pallas_call(
        _k4_body,
        grid=(grid,),
        in_specs=[
            pl.BlockSpec((blk, MW), row),
            pl.BlockSpec((blk, MW), row),
        ],
        out_specs=[
            pl.BlockSpec((blk, D), row),
            pl.BlockSpec((blk, 3 * D), row),
            pl.BlockSpec((blk, 9 * D), row),
        ],
        out_shape=[
            jax.ShapeDtypeStruct((N, D), jnp.float32),
            jax.ShapeDtypeStruct((N, 3 * D), jnp.float32),
            jax.ShapeDtypeStruct((N, 9 * D), jnp.float32),
        ],
    )(acc0, acc1)


def kernel(ind_2, p1, p3, p5, diff, basis, pp1_w1, pp1_b1, pp1_w2, pp1_b2,
           pi1_w1, pi1_b1, pi1_w2, pi1_b2, ii1_w1, ii1_w2, pp3_w1, pp3_w2,
           ii3_w1, ii3_w2, pp5_w1, pp5_b1, pp5_w2, pp5_b2, ii5_w1, ii5_w2):
    f32 = jnp.float32
    eye3 = jnp.eye(3, dtype=f32)
    eye9 = jnp.eye(9, dtype=f32)

    wa = pi1_w1[:D]
    wb = pi1_w1[D:]
    w3f = pp3_w1 @ pp3_w2 @ ii3_w1
    w3a = jnp.kron(eye3, w3f)
    w3b = jnp.kron(eye3, ii3_w2)
    w5a = jnp.kron(eye9, pp5_w1)
    b5a = jnp.tile(pp5_b1, 9)[None, :]
    w5b = jnp.kron(eye9, pp5_w2)
    b5b = jnp.tile(pp5_b2, 9)[None, :]
    w5c = jnp.kron(eye9, ii5_w1)
    w5d = jnp.kron(eye9, ii5_w2)
    w1f = jnp.repeat(ii1_w1, NB, axis=0)

    b11 = pp1_b1[None, :]
    b12 = pp1_b2[None, :]
    b1i = pi1_b1[None, :]
    b2 = pi1_b2[None, :]

    p3f = p3.reshape(N, 3 * D)
    p5f = p5.reshape(N, 9 * D)

    ta, tt = _node_precompute(p1, p3f, p5f, pp1_w1, b11, pp1_w2, b12, wa, wb,
                              b1i, w3a, w3b, w5a, b5a, w5b, b5b, w5c, w5d)

    pad = E_PAD - E
    ind_i = ind_2[:, 0]
    ind_j = ind_2[:, 1]
    indi = jnp.concatenate([ind_i, jnp.zeros((pad,), jnp.int32)])
    indj = jnp.concatenate([ind_j, jnp.zeros((pad,), jnp.int32)])
    inds = jnp.concatenate([ind_i, jnp.full((pad,), N, jnp.int32)])
    indi2d = indi.reshape(E_PAD // CH, CH)
    indj2d = indj.reshape(E_PAD // CH, CH)
    inds2d = inds.reshape(E_PAD // CH, CH)
    dp = jnp.concatenate([diff, basis, jnp.zeros((E, 3), f32)], axis=1)
    dp = jnp.concatenate([dp, jnp.zeros((pad, D), f32)], axis=0)

    gt = _sc_gather(ta, tt, indi2d, indj2d)
    m0, m1 = _edge_mlp(gt, dp, pi1_w2, b2, w1f, ii1_w2)
    zeros = jnp.zeros((N_PAD, MW), f32)
    acc = _sc_scatter(m0, m1, inds2d, zeros)
    o1, o3, o5 = _combine(acc[:N], acc[N_PAD:N_PAD + N])
    return (o1, o3.reshape(N, 3, D), o5.reshape(N, 3, 3, D))
